# baseline (speedup 1.0000x reference)
import functools

import jax
import jax.numpy as jnp
from jax import lax
from jax.experimental import pallas as pl
from jax.experimental.pallas import tpu as pltpu
from jax.experimental.pallas import tpu_sc as plsc

N_USERS = 1000000
N_ITEMS = 100000
F = 32
BATCH = 16384

NC = 2
NS = 16
NW = NC * NS
BPW = BATCH // NW
CHUNK = 128
NCHUNK = BPW // CHUNK
L = 16
SPAD = 17


def _body(user_hbm, item_hbm, uf_hbm, if_hbm, out_hbm,
          uidx, iidx, urows, vrows, sbuf, oloc, sem):
    wid = lax.axis_index("s") * NC + lax.axis_index("c")
    base = wid * BPW

    for j in range(NCHUNK):
        pltpu.sync_copy(user_hbm.at[pl.ds(base + j * CHUNK, CHUNK)], uidx.at[j])
        pltpu.sync_copy(item_hbm.at[pl.ds(base + j * CHUNK, CHUNK)], iidx.at[j])

    copies = []
    for j in range(NCHUNK):
        copies.append(pltpu.async_copy(
            uf_hbm.at[uidx.at[j]], urows.at[pl.ds(j * CHUNK, CHUNK)], sem))
        copies.append(pltpu.async_copy(
            if_hbm.at[iidx.at[j]], vrows.at[pl.ds(j * CHUNK, CHUNK)], sem))
    for c in copies:
        c.wait()

    def mul_fold(r, _):
        u0 = urows[r, pl.ds(0, L)]
        u1 = urows[r, pl.ds(L, L)]
        v0 = vrows[r, pl.ds(0, L)]
        v1 = vrows[r, pl.ds(L, L)]
        sbuf[pl.ds(r * SPAD, L)] = u0 * v0 + u1 * v1
        return 0

    lax.fori_loop(0, BPW, mul_fold, 0)

    lanes = lax.iota(jnp.int32, L)

    def reduce_group(g, _):
        flat = (g * L + lanes) * SPAD
        acc = plsc.load_gather(sbuf, [flat])
        for j in range(1, L):
            acc = acc + plsc.load_gather(sbuf, [flat + j])
        oloc[pl.ds(g * L, L)] = acc
        return 0

    lax.fori_loop(0, BPW // L, reduce_group, 0)

    pltpu.sync_copy(oloc, out_hbm.at[pl.ds(base, BPW)])


@jax.jit
def _mf_scores(user, item, user_factors, item_factors):
    mesh = plsc.VectorSubcoreMesh(core_axis_name="c", subcore_axis_name="s")
    kfn = functools.partial(
        pl.kernel,
        out_type=jax.ShapeDtypeStruct((BATCH,), jnp.float32),
        mesh=mesh,
        compiler_params=pltpu.CompilerParams(
            needs_layout_passes=False, use_tc_tiling_on_sc=False),
        scratch_types=[
            pltpu.VMEM((NCHUNK, CHUNK), jnp.int32),
            pltpu.VMEM((NCHUNK, CHUNK), jnp.int32),
            pltpu.VMEM((BPW, F), jnp.float32),
            pltpu.VMEM((BPW, F), jnp.float32),
            pltpu.VMEM((BPW * SPAD,), jnp.float32),
            pltpu.VMEM((BPW,), jnp.float32),
            pltpu.SemaphoreType.DMA,
        ],
    )(_body)
    return kfn(user, item, user_factors, item_factors)


def kernel(user, item, user_factors, item_factors):
    return _mf_scores(user.astype(jnp.int32), item.astype(jnp.int32),
                      user_factors, item_factors)
